# Initial kernel scaffold; baseline (speedup 1.0000x reference)
import jax
import jax.numpy as jnp
from jax.experimental import pallas as pl


def kernel(attention_scores, rel_bias_table, math_bias_scale, math_pattern_table):
    raise NotImplementedError("write your pallas kernel here")


# speedup vs baseline: 79.7142x; 79.7142x over previous
import functools
import jax
import jax.numpy as jnp
from jax.experimental import pallas as pl
from jax.experimental.pallas import tpu as pltpu

H = 12
Q = 2048
K = 2048
NB = 32
SKEW = 8
LJ = 4224
TQ = 256


def _bucket_from_d(d):
    pos = (d > 0).astype(jnp.int32) * 16
    a = jnp.abs(d)
    a2f = (a * a).astype(jnp.float32)
    e = (jax.lax.bitcast_convert_type(a2f, jnp.int32) >> 23) - 127
    large = jnp.minimum(2 + e, 15)
    return pos + jnp.where(a < 8, a, large)


def _stable_body(ct2_ref, s_ref):
    rows = jax.lax.broadcasted_iota(jnp.int32, (96, LJ), 0)
    j = jax.lax.broadcasted_iota(jnp.int32, (96, LJ), 1)
    d = j - (rows % SKEW) - (Q - 1)
    bucket = _bucket_from_d(d)
    acc = jnp.zeros((96, LJ), jnp.float32)
    for b in range(NB):
        acc = acc + (bucket == b).astype(jnp.float32) * ct2_ref[:, b : b + 1]
    s_ref[:, :] = acc


def _build_s_table(ct2):
    return pl.pallas_call(
        _stable_body,
        out_shape=jax.ShapeDtypeStruct((96, LJ), jnp.float32),
    )(ct2)


def _add_body(attn_ref, s_ref, out_ref):
    q0 = pl.program_id(1) * TQ

    def g_body(g, carry):
        c0 = (Q - 1) - q0 - SKEW * g
        rolled = pltpu.roll(s_ref[:, :], LJ - c0, axis=1)
        bias = rolled[:, :K]
        r0 = SKEW * g
        out_ref[0, pl.ds(r0, SKEW), :] = attn_ref[0, pl.ds(r0, SKEW), :] + bias
        return carry

    jax.lax.fori_loop(0, TQ // SKEW, g_body, 0)


def _bias_add(attn, s2):
    return pl.pallas_call(
        _add_body,
        grid=(H, Q // TQ),
        in_specs=[
            pl.BlockSpec((1, TQ, K), lambda h, qi: (h, qi, 0)),
            pl.BlockSpec((SKEW, LJ), lambda h, qi: (h, 0)),
        ],
        out_specs=pl.BlockSpec((1, TQ, K), lambda h, qi: (h, qi, 0)),
        out_shape=jax.ShapeDtypeStruct((H, Q, K), jnp.float32),
    )(attn, s2)


@jax.jit
def kernel(attention_scores, rel_bias_table, math_bias_scale, math_pattern_table):
    combined = (
        rel_bias_table * math_bias_scale[None, :] + math_pattern_table[0][None, :]
    )
    ct2 = jnp.repeat(combined.T, SKEW, axis=0)
    s2 = _build_s_table(ct2)
    out = _bias_add(attention_scores[0], s2)
    return out[None]


# speedup vs baseline: 140.6895x; 1.7649x over previous
import functools
import jax
import jax.numpy as jnp
from jax.experimental import pallas as pl
from jax.experimental.pallas import tpu as pltpu

H = 12
Q = 2048
K = 2048
NB = 32
SKEW = 128
OFF = 2048
LJ = 4224
TQ = 128


def _bucket_from_d(d):
    pos = (d > 0).astype(jnp.int32) * 16
    a = jnp.abs(d)
    a2f = (a * a).astype(jnp.float32)
    e = (jax.lax.bitcast_convert_type(a2f, jnp.int32) >> 23) - 127
    large = jnp.minimum(2 + e, 15)
    return pos + jnp.where(a < 8, a, large)


def _stable_body(ct2_ref, s_ref):
    rows = jax.lax.broadcasted_iota(jnp.int32, (8, LJ), 0)
    j = jax.lax.broadcasted_iota(jnp.int32, (8, LJ), 1)
    d = j - rows - OFF
    bucket = _bucket_from_d(d)
    acc = jnp.zeros((8, LJ), jnp.float32)
    for b in range(NB):
        acc = acc + (bucket == b).astype(jnp.float32) * ct2_ref[:, b : b + 1]
    for t in range(SKEW // 8):
        s_ref[8 * t : 8 * t + 8, :] = pltpu.roll(acc, 8 * t, axis=1)


def _build_s_table(ct2):
    return pl.pallas_call(
        _stable_body,
        grid=(H,),
        in_specs=[pl.BlockSpec((8, NB), lambda h: (h, 0))],
        out_specs=pl.BlockSpec((SKEW, LJ), lambda h: (h, 0)),
        out_shape=jax.ShapeDtypeStruct((H * SKEW, LJ), jnp.float32),
    )(ct2)


def _add_body(attn_ref, s_ref, out_ref):
    m = pl.program_id(1)
    c0 = pl.multiple_of(SKEW * (Q // TQ - m), SKEW)
    out_ref[0] = attn_ref[0] + s_ref[:, pl.ds(c0, K)]


def _bias_add(attn, s128):
    return pl.pallas_call(
        _add_body,
        grid=(H, Q // TQ),
        in_specs=[
            pl.BlockSpec((1, TQ, K), lambda h, m: (h, m, 0)),
            pl.BlockSpec((SKEW, LJ), lambda h, m: (h, 0)),
        ],
        out_specs=pl.BlockSpec((1, TQ, K), lambda h, m: (h, m, 0)),
        out_shape=jax.ShapeDtypeStruct((H, Q, K), jnp.float32),
    )(attn, s128)


@jax.jit
def kernel(attention_scores, rel_bias_table, math_bias_scale, math_pattern_table):
    combined = (
        rel_bias_table * math_bias_scale[None, :] + math_pattern_table[0][None, :]
    )
    ct2 = jnp.repeat(combined.T, 8, axis=0)
    s128 = _build_s_table(ct2)
    out = _bias_add(attention_scores[0], s128)
    return out[None]
